# baseline (speedup 1.0000x reference)
import functools

import jax
import jax.numpy as jnp
from jax import lax
from jax.experimental import pallas as pl
from jax.experimental.pallas import tpu as pltpu
from jax.experimental.pallas import tpu_sc as plsc

L = 16
NC = 2
NS = 16
NW = NC * NS
ROWS = 50
COLS = 200
NCHUNK = (COLS + L - 1) // L
COLS_PAD = NCHUNK * L
ROWS_PER_W = 2
TABLE = 1000


def _sc_body(hist_hbm, cur_hbm, table_hbm, out_hbm, tail_hbm,
             hist_v, cur_v, row_a, row_b, e_a, e_b,
             hsem, csem, asem, bsem, osem):
    cid = lax.axis_index("c")
    sid = lax.axis_index("s")
    wid = sid * NC + cid

    hcopy = pltpu.async_copy(hist_hbm, hist_v.at[pl.ds(0, COLS)], hsem)
    pltpu.async_copy(cur_hbm, cur_v.at[pl.ds(0, ROWS)], csem).wait()

    i0 = jnp.minimum(wid * ROWS_PER_W, ROWS - ROWS_PER_W)
    i1 = i0 + 1
    curpair = cur_v[pl.ds(i0, L)]
    r0 = curpair[0]
    r1 = curpair[1]
    acopy = pltpu.async_copy(table_hbm.at[r0], row_a, asem)
    bcopy = pltpu.async_copy(table_hbm.at[r1], row_b, bsem)
    hcopy.wait()

    lane = lax.broadcasted_iota(jnp.int32, (L,), 0)
    OVERLAP = NCHUNK * L - COLS
    NFULL = NCHUNK - 1

    out_copies = []
    for i_out, row_v, e_v, cp in ((i0, row_a, e_a, acopy),
                                  (i1, row_b, e_b, bcopy)):
        cp.wait()
        e_v[pl.ds(NFULL * L, L)] = jnp.zeros((L,), jnp.float32)
        vtail = plsc.load_gather(row_v, [hist_v[pl.ds(COLS - L, L)]])
        vtail = jnp.where(lane >= OVERLAP, jnp.exp(vtail), 0.0)
        e_v[pl.ds(COLS - L, L)] = vtail

        def p1(c, s):
            t = jnp.exp(plsc.load_gather(row_v, [hist_v[pl.ds(c * L, L)]]))
            e_v[pl.ds(c * L, L)] = t
            return s + t
        s = lax.fori_loop(0, NFULL, p1, vtail)
        inv = jnp.full((L,), 1.0, jnp.float32) / jnp.broadcast_to(
            jnp.sum(s), (L,))

        def p2(c, carry):
            e_v[pl.ds(c * L, L)] = e_v[pl.ds(c * L, L)] * inv
            return carry
        lax.fori_loop(0, NCHUNK, p2, jnp.int32(0))

        out_copies.append(
            pltpu.async_copy(e_v.at[pl.ds(0, 128)],
                             out_hbm.at[i_out, pl.ds(0, 128)], osem))
        out_copies.append(
            pltpu.async_copy(e_v.at[pl.ds(128, 128)],
                             tail_hbm.at[i_out, pl.ds(0, 128)], osem))

    for cp in out_copies:
        cp.wait()


@jax.jit
def _run(history, current, cid_time):
    mesh = plsc.VectorSubcoreMesh(
        core_axis_name="c", subcore_axis_name="s",
        num_cores=NC, num_subcores=NS)
    fn = pl.kernel(
        _sc_body,
        out_type=(jax.ShapeDtypeStruct((ROWS, COLS), jnp.float32),
                  jax.ShapeDtypeStruct((ROWS, 128), jnp.float32)),
        mesh=mesh,
        compiler_params=pltpu.CompilerParams(
            needs_layout_passes=False,
        ),
        scratch_types=[
            pltpu.VMEM((COLS_PAD,), jnp.int32),
            pltpu.VMEM((ROWS - ROWS_PER_W + L,), jnp.int32),
            pltpu.VMEM((TABLE,), jnp.float32),
            pltpu.VMEM((TABLE,), jnp.float32),
            pltpu.VMEM((256,), jnp.float32),
            pltpu.VMEM((256,), jnp.float32),
            pltpu.SemaphoreType.DMA,
            pltpu.SemaphoreType.DMA,
            pltpu.SemaphoreType.DMA,
            pltpu.SemaphoreType.DMA,
            pltpu.SemaphoreType.DMA,
        ],
    )
    out, tails = fn(history, current, cid_time)
    return lax.dynamic_update_slice(
        out, tails[:, :COLS - 128], (0, 128))


def kernel(history, current, cid_time):
    return _run(history.astype(jnp.int32), current.astype(jnp.int32),
                cid_time)


# speedup vs baseline: 1.0038x; 1.0038x over previous
import functools

import jax
import jax.numpy as jnp
from jax import lax
from jax.experimental import pallas as pl
from jax.experimental.pallas import tpu as pltpu
from jax.experimental.pallas import tpu_sc as plsc

L = 16
NC = 2
NS = 16
NW = NC * NS
ROWS = 50
COLS = 200
NCHUNK = (COLS + L - 1) // L
COLS_PAD = NCHUNK * L
ROWS_PER_W = 2
TABLE = 1000


def _sc_body(hist_hbm, cur_hbm, table_hbm, out_hbm, tail_hbm,
             hist_v, cur_v, row_a, row_b, e_a, e_b,
             hsem, csem, asem, bsem, osem):
    cid = lax.axis_index("c")
    sid = lax.axis_index("s")
    wid = sid * NC + cid

    hcopy = pltpu.async_copy(hist_hbm, hist_v.at[pl.ds(0, COLS)], hsem)
    pltpu.async_copy(cur_hbm, cur_v.at[pl.ds(0, ROWS)], csem).wait()

    i0 = jnp.minimum(wid * ROWS_PER_W, ROWS - ROWS_PER_W)
    i1 = i0 + 1
    curpair = cur_v[pl.ds(i0, L)]
    r0 = curpair[0]
    r1 = curpair[1]
    acopy = pltpu.async_copy(table_hbm.at[r0], row_a, asem)
    bcopy = pltpu.async_copy(table_hbm.at[r1], row_b, bsem)
    hcopy.wait()

    lane = lax.broadcasted_iota(jnp.int32, (L,), 0)
    OVERLAP = NCHUNK * L - COLS
    NFULL = NCHUNK - 1

    out_copies = []
    for i_out, row_v, e_v, cp in ((i0, row_a, e_a, acopy),
                                  (i1, row_b, e_b, bcopy)):
        cp.wait()
        e_v[pl.ds(NFULL * L, L)] = jnp.zeros((L,), jnp.float32)
        vtail = plsc.load_gather(row_v, [hist_v[pl.ds(COLS - L, L)]])
        vtail = jnp.where(lane >= OVERLAP, jnp.exp(vtail), 0.0)
        e_v[pl.ds(COLS - L, L)] = vtail

        def p1(c, s):
            t = jnp.exp(plsc.load_gather(row_v, [hist_v[pl.ds(c * L, L)]]))
            e_v[pl.ds(c * L, L)] = t
            return s + t
        s = lax.fori_loop(0, NFULL, p1, vtail)
        inv = jnp.full((L,), 1.0, jnp.float32) / jnp.broadcast_to(
            jnp.sum(s), (L,))

        def p2(c, carry):
            e_v[pl.ds(c * L, L)] = e_v[pl.ds(c * L, L)] * inv
            return carry
        lax.fori_loop(0, NCHUNK, p2, jnp.int32(0))

        out_copies.append(
            pltpu.async_copy(e_v.at[pl.ds(0, 128)],
                             out_hbm.at[i_out, pl.ds(0, 128)], osem))
        out_copies.append(
            pltpu.async_copy(e_v.at[pl.ds(128, 128)],
                             tail_hbm.at[i_out, pl.ds(0, 128)], osem))

    for cp in out_copies:
        cp.wait()


@jax.jit
def _run(history, current, cid_time):
    mesh = plsc.VectorSubcoreMesh(
        core_axis_name="c", subcore_axis_name="s",
        num_cores=NC, num_subcores=NS)
    fn = pl.kernel(
        _sc_body,
        out_type=(jax.ShapeDtypeStruct((ROWS, COLS), jnp.float32),
                  jax.ShapeDtypeStruct((ROWS, 128), jnp.float32)),
        mesh=mesh,
        compiler_params=pltpu.CompilerParams(
            needs_layout_passes=False,
            skip_device_barrier=True,
        ),
        scratch_types=[
            pltpu.VMEM((COLS_PAD,), jnp.int32),
            pltpu.VMEM((ROWS - ROWS_PER_W + L,), jnp.int32),
            pltpu.VMEM((TABLE,), jnp.float32),
            pltpu.VMEM((TABLE,), jnp.float32),
            pltpu.VMEM((256,), jnp.float32),
            pltpu.VMEM((256,), jnp.float32),
            pltpu.SemaphoreType.DMA,
            pltpu.SemaphoreType.DMA,
            pltpu.SemaphoreType.DMA,
            pltpu.SemaphoreType.DMA,
            pltpu.SemaphoreType.DMA,
        ],
    )
    out, tails = fn(history, current, cid_time)
    return lax.dynamic_update_slice(
        out, tails[:, :COLS - 128], (0, 128))


def kernel(history, current, cid_time):
    return _run(history.astype(jnp.int32), current.astype(jnp.int32),
                cid_time)


# speedup vs baseline: 1.0101x; 1.0063x over previous
import functools

import jax
import jax.numpy as jnp
from jax import lax
from jax.experimental import pallas as pl
from jax.experimental.pallas import tpu as pltpu
from jax.experimental.pallas import tpu_sc as plsc

L = 16
NC = 2
NS = 16
NW = NC * NS
ROWS = 50
COLS = 200
NCHUNK = (COLS + L - 1) // L
COLS_PAD = NCHUNK * L
ROWS_PER_W = 2
TABLE = 1000


def _sc_body(hist_hbm, cur_hbm, table_hbm, out_hbm, tail_hbm,
             hist_v, cur_v, row_a, row_b, e_a, e_b,
             hsem, csem, asem, bsem, osem):
    cid = lax.axis_index("c")
    sid = lax.axis_index("s")
    wid = sid * NC + cid

    hcopy = pltpu.async_copy(hist_hbm, hist_v.at[pl.ds(0, COLS)], hsem)
    pltpu.async_copy(cur_hbm, cur_v.at[pl.ds(0, ROWS)], csem).wait()

    i0 = jnp.minimum(wid * ROWS_PER_W, ROWS - ROWS_PER_W)
    i1 = i0 + 1
    curpair = cur_v[pl.ds(i0, L)]
    r0 = curpair[0]
    r1 = curpair[1]
    acopy = pltpu.async_copy(table_hbm.at[r0], row_a, asem)
    bcopy = pltpu.async_copy(table_hbm.at[r1], row_b, bsem)
    hcopy.wait()

    lane = lax.broadcasted_iota(jnp.int32, (L,), 0)
    OVERLAP = NCHUNK * L - COLS
    NFULL = NCHUNK - 1

    out_copies = []
    for i_out, row_v, e_v, cp in ((i0, row_a, e_a, acopy),
                                  (i1, row_b, e_b, bcopy)):
        cp.wait()
        e_v[pl.ds(NFULL * L, L)] = jnp.full((L,), -jnp.inf, jnp.float32)
        vtail = plsc.load_gather(row_v, [hist_v[pl.ds(COLS - L, L)]])
        vtail = jnp.where(lane >= OVERLAP, vtail, -jnp.inf)
        e_v[pl.ds(COLS - L, L)] = vtail

        def p1(c, m):
            v = plsc.load_gather(row_v, [hist_v[pl.ds(c * L, L)]])
            e_v[pl.ds(c * L, L)] = v
            return jnp.maximum(m, v)
        m = lax.fori_loop(0, NFULL, p1, vtail)
        mmax = jnp.max(m)

        def p2(c, s):
            t = jnp.exp(e_v[pl.ds(c * L, L)] - mmax)
            e_v[pl.ds(c * L, L)] = t
            return s + t
        s = lax.fori_loop(0, NCHUNK, p2, jnp.zeros((L,), jnp.float32))
        inv = jnp.full((L,), 1.0, jnp.float32) / jnp.broadcast_to(
            jnp.sum(s), (L,))

        def p3(c, carry):
            e_v[pl.ds(c * L, L)] = e_v[pl.ds(c * L, L)] * inv
            return carry
        lax.fori_loop(0, NCHUNK, p3, jnp.int32(0))

        out_copies.append(
            pltpu.async_copy(e_v.at[pl.ds(0, 128)],
                             out_hbm.at[i_out, pl.ds(0, 128)], osem))
        out_copies.append(
            pltpu.async_copy(e_v.at[pl.ds(128, 128)],
                             tail_hbm.at[i_out, pl.ds(0, 128)], osem))

    for cp in out_copies:
        cp.wait()


@jax.jit
def _run(history, current, cid_time):
    mesh = plsc.VectorSubcoreMesh(
        core_axis_name="c", subcore_axis_name="s",
        num_cores=NC, num_subcores=NS)
    fn = pl.kernel(
        _sc_body,
        out_type=(jax.ShapeDtypeStruct((ROWS, COLS), jnp.float32),
                  jax.ShapeDtypeStruct((ROWS, 128), jnp.float32)),
        mesh=mesh,
        compiler_params=pltpu.CompilerParams(
            needs_layout_passes=False,
        ),
        scratch_types=[
            pltpu.VMEM((COLS_PAD,), jnp.int32),
            pltpu.VMEM((ROWS - ROWS_PER_W + L,), jnp.int32),
            pltpu.VMEM((TABLE,), jnp.float32),
            pltpu.VMEM((TABLE,), jnp.float32),
            pltpu.VMEM((256,), jnp.float32),
            pltpu.VMEM((256,), jnp.float32),
            pltpu.SemaphoreType.DMA,
            pltpu.SemaphoreType.DMA,
            pltpu.SemaphoreType.DMA,
            pltpu.SemaphoreType.DMA,
            pltpu.SemaphoreType.DMA,
        ],
    )
    out, tails = fn(history, current, cid_time)
    return lax.dynamic_update_slice(
        out, tails[:, :COLS - 128], (0, 128))


def kernel(history, current, cid_time):
    return _run(history.astype(jnp.int32), current.astype(jnp.int32),
                cid_time)


# speedup vs baseline: 1.0473x; 1.0369x over previous
import jax
import jax.numpy as jnp
from jax import lax
from jax.experimental import pallas as pl
from jax.experimental.pallas import tpu as pltpu
from jax.experimental.pallas import tpu_sc as plsc

L = 16
NC = 1
NS = 16
NW = NC * NS
ROWS = 50
COLS = 200
NCHUNK = (COLS + L - 1) // L
COLS_PAD = NCHUNK * L
ROWS_PER_W = 4
TABLE = 1000


def _sc_body(hist_hbm, cur_hbm, table_hbm, out_hbm, tail_hbm,
             hist_v, cur_v,
             row_0, row_1, row_2, row_3, e_0, e_1, e_2, e_3,
             hsem, csem, s0, s1, s2, s3, osem):
    cid = lax.axis_index("c")
    sid = lax.axis_index("s")
    wid = sid * NC + cid

    rows = (row_0, row_1, row_2, row_3)
    es = (e_0, e_1, e_2, e_3)
    sems = (s0, s1, s2, s3)

    hcopy = pltpu.async_copy(hist_hbm, hist_v.at[pl.ds(0, COLS)], hsem)
    pltpu.async_copy(cur_hbm, cur_v.at[pl.ds(0, ROWS)], csem).wait()

    i0 = jnp.minimum(wid * ROWS_PER_W, ROWS - ROWS_PER_W)
    curvec = cur_v[pl.ds(i0, L)]
    copies = [
        pltpu.async_copy(table_hbm.at[curvec[r]], rows[r], sems[r])
        for r in range(ROWS_PER_W)
    ]
    hcopy.wait()

    lane = lax.broadcasted_iota(jnp.int32, (L,), 0)
    OVERLAP = NCHUNK * L - COLS
    NFULL = NCHUNK - 1

    out_copies = []
    for r in range(ROWS_PER_W):
        i_out = i0 + r
        row_v, e_v, cp = rows[r], es[r], copies[r]
        cp.wait()
        e_v[pl.ds(NFULL * L, L)] = jnp.full((L,), -jnp.inf, jnp.float32)
        vtail = plsc.load_gather(row_v, [hist_v[pl.ds(COLS - L, L)]])
        vtail = jnp.where(lane >= OVERLAP, vtail, -jnp.inf)
        e_v[pl.ds(COLS - L, L)] = vtail

        def p1(c, m):
            v = plsc.load_gather(row_v, [hist_v[pl.ds(c * L, L)]])
            e_v[pl.ds(c * L, L)] = v
            return jnp.maximum(m, v)
        m = lax.fori_loop(0, NFULL, p1, vtail)
        mmax = jnp.max(m)

        def p2(c, s):
            t = jnp.exp(e_v[pl.ds(c * L, L)] - mmax)
            e_v[pl.ds(c * L, L)] = t
            return s + t
        s = lax.fori_loop(0, NCHUNK, p2, jnp.zeros((L,), jnp.float32))
        inv = jnp.full((L,), 1.0, jnp.float32) / jnp.broadcast_to(
            jnp.sum(s), (L,))

        def p3(c, carry):
            e_v[pl.ds(c * L, L)] = e_v[pl.ds(c * L, L)] * inv
            return carry
        lax.fori_loop(0, NCHUNK, p3, jnp.int32(0))

        out_copies.append(
            pltpu.async_copy(e_v.at[pl.ds(0, 128)],
                             out_hbm.at[i_out, pl.ds(0, 128)], osem))
        out_copies.append(
            pltpu.async_copy(e_v.at[pl.ds(128, 128)],
                             tail_hbm.at[i_out, pl.ds(0, 128)], osem))

    for cp in out_copies:
        cp.wait()


@jax.jit
def _run(history, current, cid_time):
    mesh = plsc.VectorSubcoreMesh(
        core_axis_name="c", subcore_axis_name="s",
        num_cores=NC, num_subcores=NS)
    fn = pl.kernel(
        _sc_body,
        out_type=(jax.ShapeDtypeStruct((ROWS, COLS), jnp.float32),
                  jax.ShapeDtypeStruct((ROWS, 128), jnp.float32)),
        mesh=mesh,
        compiler_params=pltpu.CompilerParams(
            needs_layout_passes=False,
        ),
        scratch_types=(
            [pltpu.VMEM((COLS_PAD,), jnp.int32),
             pltpu.VMEM((ROWS - ROWS_PER_W + L,), jnp.int32)]
            + [pltpu.VMEM((TABLE,), jnp.float32)] * ROWS_PER_W
            + [pltpu.VMEM((256,), jnp.float32)] * ROWS_PER_W
            + [pltpu.SemaphoreType.DMA] * (ROWS_PER_W + 3)
        ),
    )
    out, tails = fn(history, current, cid_time)
    return lax.dynamic_update_slice(
        out, tails[:, :COLS - 128], (0, 128))


def kernel(history, current, cid_time):
    return _run(history.astype(jnp.int32), current.astype(jnp.int32),
                cid_time)
